# baseline (speedup 1.0000x reference)
import jax
import jax.numpy as jnp
from jax.experimental import pallas as pl
from jax.experimental.pallas import tpu as pltpu

N_CHANNELS = 4096
N_BINS = 8192
BLOCK_ROWS = 256
NBLK = N_CHANNELS // BLOCK_ROWS
NBUF = 6


def _body(params_ref, energies_ref, tm_ref, out_ref, buf_ref, flux_ref, sems):
    def copy(i):
        slot = i % NBUF
        return pltpu.make_async_copy(
            tm_ref.at[i * BLOCK_ROWS:(i + 1) * BLOCK_ROWS, :],
            buf_ref.at[slot],
            sems.at[slot],
        )

    for i in range(NBUF):
        copy(i).start()

    alpha = params_ref[0]
    norm = params_ref[1]
    p = 1.0 - alpha
    e_low = energies_ref[0:1, :]
    e_high = energies_ref[1:2, :]
    flux_ref[...] = norm * (jnp.power(e_high, p) - jnp.power(e_low, p)) / p

    for i in range(NBLK):
        copy(i).wait()
        tile = buf_ref[i % NBUF]
        acc = jnp.sum(tile * flux_ref[...], axis=1, keepdims=True)
        out_ref[i * BLOCK_ROWS:(i + 1) * BLOCK_ROWS, :] = jnp.maximum(acc, 1e-6)
        if i + NBUF < NBLK:
            copy(i + NBUF).start()


def kernel(parameters, transfer_matrix, energies):
    out = pl.pallas_call(
        _body,
        in_specs=[
            pl.BlockSpec(memory_space=pltpu.MemorySpace.SMEM),
            pl.BlockSpec(memory_space=pltpu.MemorySpace.VMEM),
            pl.BlockSpec(memory_space=pltpu.MemorySpace.HBM),
        ],
        out_specs=pl.BlockSpec(memory_space=pltpu.MemorySpace.VMEM),
        out_shape=jax.ShapeDtypeStruct((N_CHANNELS, 1), jnp.float32),
        scratch_shapes=[
            pltpu.MemorySpace.VMEM((NBUF, BLOCK_ROWS, N_BINS), jnp.float32),
            pltpu.MemorySpace.VMEM((1, N_BINS), jnp.float32),
            pltpu.SemaphoreType.DMA((NBUF,)),
        ],
    )(parameters, energies, transfer_matrix)
    return out.reshape(N_CHANNELS)


# speedup vs baseline: 1.0880x; 1.0880x over previous
import jax
import jax.numpy as jnp
from jax.experimental import pallas as pl
from jax.experimental.pallas import tpu as pltpu

N_CHANNELS = 4096
N_BINS = 8192
BLOCK_ROWS = 256


def _body(params_ref, energies_ref, tm_ref, out_ref, flux_ref):
    @pl.when(pl.program_id(0) == 0)
    def _():
        alpha = params_ref[0]
        norm = params_ref[1]
        p = 1.0 - alpha
        e_low = energies_ref[0:1, :]
        e_high = energies_ref[1:2, :]
        flux_ref[...] = norm * (jnp.power(e_high, p) - jnp.power(e_low, p)) / p

    tile = tm_ref[...]
    acc = jax.lax.dot_general(
        tile, flux_ref[...],
        (((1,), (1,)), ((), ())),
        preferred_element_type=jnp.float32,
    )
    out_ref[...] = jnp.maximum(acc, 1e-6)


def kernel(parameters, transfer_matrix, energies):
    grid = (N_CHANNELS // BLOCK_ROWS,)
    out = pl.pallas_call(
        _body,
        grid=grid,
        in_specs=[
            pl.BlockSpec(memory_space=pltpu.MemorySpace.SMEM),
            pl.BlockSpec((2, N_BINS), lambda i: (0, 0)),
            pl.BlockSpec((BLOCK_ROWS, N_BINS), lambda i: (i, 0)),
        ],
        out_specs=pl.BlockSpec((BLOCK_ROWS, 1), lambda i: (i, 0)),
        out_shape=jax.ShapeDtypeStruct((N_CHANNELS, 1), jnp.float32),
        scratch_shapes=[pltpu.MemorySpace.VMEM((1, N_BINS), jnp.float32)],
    )(parameters, energies, transfer_matrix)
    return out.reshape(N_CHANNELS)
